# Initial kernel scaffold; baseline (speedup 1.0000x reference)
import jax
import jax.numpy as jnp
from jax.experimental import pallas as pl


def kernel(input, rim_hidden_states, hidden_to_query_map, input_to_key_map, input_to_values_map, w_ih, w_hh):
    raise NotImplementedError("write your pallas kernel here")


# speedup vs baseline: 2.0892x; 2.0892x over previous
import jax
import jax.numpy as jnp
from jax.experimental import pallas as pl

ACTIVE_KERNELS = 4


def _rim_body(x_ref, h_ref, wq_ref, wk_ref, wv_ref, wih_ref, whh_ref, out_ref):
    x = x_ref[0]
    h = h_ref[0]
    wq = wq_ref[...]
    wk = wk_ref[...]
    wv = wv_ref[...]
    wih = wih_ref[...]
    whh = whh_ref[...]

    q = jnp.sum(h[:, :, None] * wq, axis=1)
    kq = jnp.sum(wk * q[:, None, :], axis=2)
    sim = jax.lax.dot_general(kq, x, (((1,), (1,)), ((), ())),
                              preferred_element_type=jnp.float32,
                              precision=jax.lax.Precision.HIGHEST)
    m = jnp.maximum(jnp.max(sim, axis=1), 0.0)
    p = jnp.exp(sim - m[:, None])
    denom = jnp.sum(p, axis=1) + jnp.exp(-m)
    wx = jax.lax.dot_general(p, x, (((1,), (0,)), ((), ())),
                             preferred_element_type=jnp.float32,
                             precision=jax.lax.Precision.HIGHEST)
    wx = wx / denom[:, None]
    attended = jnp.sum(wx[:, :, None] * wv, axis=1)
    pre = jnp.sum(attended[:, :, None] * wih, axis=1) \
        + jnp.sum(h[:, :, None] * whh, axis=1)
    new_h = jnp.tanh(pre)
    k_idx = jax.lax.broadcasted_iota(jnp.int32, new_h.shape, 0)
    out_ref[0] = jnp.where(k_idx < ACTIVE_KERNELS, new_h, h)


def kernel(input, rim_hidden_states, hidden_to_query_map, input_to_key_map,
           input_to_values_map, w_ih, w_hh, interpret=False):
    B, S, D = input.shape
    K, H = rim_hidden_states.shape[1], rim_hidden_states.shape[2]
    A = hidden_to_query_map.shape[2]

    return pl.pallas_call(
        _rim_body,
        grid=(B,),
        in_specs=[
            pl.BlockSpec((1, S, D), lambda b: (b, 0, 0)),
            pl.BlockSpec((1, K, H), lambda b: (b, 0, 0)),
            pl.BlockSpec((K, H, A), lambda b: (0, 0, 0)),
            pl.BlockSpec((K, D, A), lambda b: (0, 0, 0)),
            pl.BlockSpec((K, D, A), lambda b: (0, 0, 0)),
            pl.BlockSpec((K, A, H), lambda b: (0, 0, 0)),
            pl.BlockSpec((K, H, H), lambda b: (0, 0, 0)),
        ],
        out_specs=pl.BlockSpec((1, K, H), lambda b: (b, 0, 0)),
        out_shape=jax.ShapeDtypeStruct((B, K, H), jnp.float32),
        interpret=interpret,
    )(input, rim_hidden_states, hidden_to_query_map, input_to_key_map,
      input_to_values_map, w_ih, w_hh)


# speedup vs baseline: 2.3770x; 1.1377x over previous
import jax
import jax.numpy as jnp
from jax.experimental import pallas as pl

ACTIVE_KERNELS = 4


def _rim_body(x_ref, h_ref, wq_ref, wk_ref, wv_ref, wih_ref, whh_ref, out_ref):
    x = x_ref[0]
    h = h_ref[0]
    wq = wq_ref[...]
    wk = wk_ref[...]
    wv = wv_ref[...]
    wih = wih_ref[...]
    whh = whh_ref[...]

    q = jnp.sum(h[:, :, None] * wq, axis=1)
    kq = jnp.sum(wk * q[:, None, :], axis=2)
    sim = jax.lax.dot_general(kq, x, (((1,), (1,)), ((), ())),
                              preferred_element_type=jnp.float32,
                              precision=jax.lax.Precision.HIGHEST)
    m = jnp.maximum(jnp.max(sim, axis=1), 0.0)
    p = jnp.exp(sim - m[:, None])
    denom = jnp.sum(p, axis=1) + jnp.exp(-m)
    wx = jax.lax.dot_general(p, x, (((1,), (0,)), ((), ())),
                             preferred_element_type=jnp.float32)
    wx = wx / denom[:, None]
    attended = jnp.sum(wx[:, :, None] * wv, axis=1)
    pre = jnp.sum(attended[:, :, None] * wih, axis=1) \
        + jnp.sum(h[:, :, None] * whh, axis=1)
    new_h = jnp.tanh(pre)
    k_idx = jax.lax.broadcasted_iota(jnp.int32, new_h.shape, 0)
    out_ref[0] = jnp.where(k_idx < ACTIVE_KERNELS, new_h, h)


def kernel(input, rim_hidden_states, hidden_to_query_map, input_to_key_map,
           input_to_values_map, w_ih, w_hh, interpret=False):
    B, S, D = input.shape
    K, H = rim_hidden_states.shape[1], rim_hidden_states.shape[2]
    A = hidden_to_query_map.shape[2]

    return pl.pallas_call(
        _rim_body,
        grid=(B,),
        in_specs=[
            pl.BlockSpec((1, S, D), lambda b: (b, 0, 0)),
            pl.BlockSpec((1, K, H), lambda b: (b, 0, 0)),
            pl.BlockSpec((K, H, A), lambda b: (0, 0, 0)),
            pl.BlockSpec((K, D, A), lambda b: (0, 0, 0)),
            pl.BlockSpec((K, D, A), lambda b: (0, 0, 0)),
            pl.BlockSpec((K, A, H), lambda b: (0, 0, 0)),
            pl.BlockSpec((K, H, H), lambda b: (0, 0, 0)),
        ],
        out_specs=pl.BlockSpec((1, K, H), lambda b: (b, 0, 0)),
        out_shape=jax.ShapeDtypeStruct((B, K, H), jnp.float32),
        interpret=interpret,
    )(input, rim_hidden_states, hidden_to_query_map, input_to_key_map,
      input_to_values_map, w_ih, w_hh)


# speedup vs baseline: 2.7464x; 1.1554x over previous
import jax
import jax.numpy as jnp
from jax.experimental import pallas as pl

ACTIVE_KERNELS = 4


def _rim_body(x_ref, h_ref, wq_ref, wk_ref, wv_ref, wih_ref, whh_ref, out_ref):
    x = x_ref[0]
    h = h_ref[0]
    wq = wq_ref[...]
    wk = wk_ref[...]
    wv = wv_ref[...]
    wih = wih_ref[...]
    whh = whh_ref[...]

    q = jnp.sum(h[:, :, None] * wq, axis=1)
    kq = jnp.sum(wk * q[:, None, :], axis=2)

    xh = x.astype(jnp.bfloat16)
    xl = (x - xh.astype(jnp.float32)).astype(jnp.bfloat16)
    kqh = kq.astype(jnp.bfloat16)
    kql = (kq - kqh.astype(jnp.float32)).astype(jnp.bfloat16)

    def dot_t(a, b):
        return jax.lax.dot_general(a, b, (((1,), (1,)), ((), ())),
                                   preferred_element_type=jnp.float32)

    def dot_s(a, b):
        return jax.lax.dot_general(a, b, (((1,), (0,)), ((), ())),
                                   preferred_element_type=jnp.float32)

    sim = dot_t(kqh, xh) + (dot_t(kqh, xl) + dot_t(kql, xh))
    m = jnp.maximum(jnp.max(sim, axis=1), 0.0)
    p = jnp.exp(sim - m[:, None])
    denom = jnp.sum(p, axis=1) + jnp.exp(-m)
    pb = p.astype(jnp.bfloat16)
    wx = dot_s(pb, xh) + dot_s(pb, xl)
    wx = wx / denom[:, None]
    attended = jnp.sum(wx[:, :, None] * wv, axis=1)
    pre = jnp.sum(attended[:, :, None] * wih, axis=1) \
        + jnp.sum(h[:, :, None] * whh, axis=1)
    new_h = jnp.tanh(pre)
    k_idx = jax.lax.broadcasted_iota(jnp.int32, new_h.shape, 0)
    out_ref[0] = jnp.where(k_idx < ACTIVE_KERNELS, new_h, h)


def kernel(input, rim_hidden_states, hidden_to_query_map, input_to_key_map,
           input_to_values_map, w_ih, w_hh, interpret=False):
    B, S, D = input.shape
    K, H = rim_hidden_states.shape[1], rim_hidden_states.shape[2]
    A = hidden_to_query_map.shape[2]

    return pl.pallas_call(
        _rim_body,
        grid=(B,),
        in_specs=[
            pl.BlockSpec((1, S, D), lambda b: (b, 0, 0)),
            pl.BlockSpec((1, K, H), lambda b: (b, 0, 0)),
            pl.BlockSpec((K, H, A), lambda b: (0, 0, 0)),
            pl.BlockSpec((K, D, A), lambda b: (0, 0, 0)),
            pl.BlockSpec((K, D, A), lambda b: (0, 0, 0)),
            pl.BlockSpec((K, A, H), lambda b: (0, 0, 0)),
            pl.BlockSpec((K, H, H), lambda b: (0, 0, 0)),
        ],
        out_specs=pl.BlockSpec((1, K, H), lambda b: (b, 0, 0)),
        out_shape=jax.ShapeDtypeStruct((B, K, H), jnp.float32),
        interpret=interpret,
    )(input, rim_hidden_states, hidden_to_query_map, input_to_key_map,
      input_to_values_map, w_ih, w_hh)
